# baseline (speedup 1.0000x reference)
import functools

import jax
import jax.numpy as jnp
from jax import lax
from jax.experimental import pallas as pl
from jax.experimental.pallas import tpu as pltpu
from jax.experimental.pallas import tpu_sc as plsc

_NC = 1
_NS = 16
_L = 16
_NW = _NC * _NS


@functools.lru_cache(maxsize=None)
def _build_sc_call(B, T, V):
    BT = B * T
    CHUNK = BT // _NW
    NCH = CHUNK // _L
    NG = CHUNK // 128
    assert BT % _NW == 0 and CHUNK % 128 == 0 and T % CHUNK == 0
    assert V % 8 == 0 and T % 128 == 0 and B % 8 == 0 and B <= _L
    NAW = CHUNK // 128 + 1
    ACT_LOAD = NAW * 128
    NTW = T // 128
    TSH = (8 * T).bit_length() - 1

    mesh = plsc.VectorSubcoreMesh(core_axis_name="c", subcore_axis_name="s",
                                  num_cores=_NC)

    @functools.partial(
        pl.kernel,
        mesh=mesh,
        compiler_params=pltpu.CompilerParams(needs_layout_passes=False),
        out_type=jax.ShapeDtypeStruct((_NW, _L), jnp.float32),
        scratch_types=[
            pltpu.VMEM((ACT_LOAD,), jnp.int32),
            pltpu.VMEM((NG, 128), jnp.int32),
            pltpu.VMEM((_L,), jnp.int32),
            pltpu.VMEM((CHUNK + _L,), jnp.float32),
            pltpu.VMEM((CHUNK + _L,), jnp.float32),
            pltpu.VMEM((_L,), jnp.float32),
            pltpu.VMEM((_L,), jnp.int32),
            pltpu.VMEM((_L,), jnp.float32),
            pltpu.VMEM((_L,), jnp.float32),
            pltpu.SemaphoreType.DMA,
            pltpu.SemaphoreType.DMA,
            pltpu.SemaphoreType.DMA,
        ],
    )
    def sc_call(matq_hbm, matt_hbm, act_hbm, sl_hbm, rw_hbm, out_hbm,
                act_v, offs_v, offx_v, q_v, qt_v, xtr_v, sl_v, rw_v,
                part_v, semq, semt, sema):
        cid = lax.axis_index("c")
        sid = lax.axis_index("s")
        wid = sid * _NC + cid
        base = wid * CHUNK
        row = base // T
        t0 = base - row * T
        iota = lax.iota(jnp.int32, _L)

        g8 = lax.shift_right_logical(row, 3)
        s8 = jnp.bitwise_and(row, 7)
        u0 = lax.shift_right_logical(t0, 7)
        acopies = []
        for k in range(NAW):
            u = jnp.minimum(u0 + k, NTW - 1)
            src = ((g8 * NTW + u) * 8 + s8) * 128
            acopies.append(pltpu.async_copy(
                act_hbm.at[pl.ds(src, 128)],
                act_v.at[pl.ds(k * 128, 128)], sema))
        pltpu.sync_copy(sl_hbm, sl_v)
        pltpu.sync_copy(rw_hbm, rw_v)
        for cp in acopies:
            cp.wait()

        rowbase = row * V

        def phys_off(t16, a16):
            r16 = rowbase + a16
            return (lax.shift_left(lax.shift_right_logical(r16, 3), TSH)
                    + lax.shift_left(lax.shift_right_logical(t16, 7), 10)
                    + lax.shift_left(jnp.bitwise_and(r16, 7), 7)
                    + jnp.bitwise_and(t16, 127))

        def mk_offs(j, _):
            a16 = act_v[pl.ds(j * _L, _L)]
            t16 = t0 + j * _L + iota
            offs_v[j // 8, pl.ds((j % 8) * _L, _L)] = phys_off(t16, a16)
            return 0
        lax.fori_loop(0, NCH, mk_offs, 0)

        a_x16 = act_v[pl.ds(CHUNK, _L)]
        t_x16 = jnp.minimum(t0 + CHUNK + iota, T - 1)
        offx_v[...] = phys_off(t_x16, a_x16)

        copies = []
        for c in range(NG):
            copies.append(pltpu.async_copy(
                matq_hbm.at[offs_v.at[c]],
                q_v.at[pl.ds(c * 128, 128)], semq))
            copies.append(pltpu.async_copy(
                matt_hbm.at[offs_v.at[c]],
                qt_v.at[pl.ds(c * 128, 128)], semt))
        copies.append(pltpu.async_copy(matt_hbm.at[offx_v], xtr_v, semt))
        for cp in copies:
            cp.wait()
        plsc.store_scatter(qt_v, [iota * 0 + CHUNK], xtr_v[...],
                           mask=iota == 0)

        row16 = iota * 0 + row
        slr = plsc.load_gather(sl_v, [row16])
        rwr = plsc.load_gather(rw_v, [row16])
        tposv = jnp.where(slr == 0, T - 1, slr - 1)

        def acc_step(j, acc, last):
            q16 = q_v[pl.ds(j * _L, _L)]
            qtn16 = qt_v[pl.ds(j * _L + 1, _L)]
            t16 = t0 + j * _L + iota
            qb = qtn16
            if last:
                qb = jnp.where(t16 == T - 1, jnp.float32(0.0), qb)
            qb = jnp.where(t16 == tposv, rwr, qb)
            qb = jnp.minimum(jnp.maximum(qb, jnp.float32(-1.0)),
                             jnp.float32(0.0))
            d = q16 - qb
            return acc + d * d

        acc = lax.fori_loop(
            0, NCH - 1, lambda j, a: acc_step(j, a, False),
            jnp.zeros((_L,), jnp.float32))
        acc = acc_step(NCH - 1, acc, True)
        part_v[...] = acc
        pltpu.sync_copy(part_v, out_hbm.at[wid])

    return sc_call


def _phys_flat(x, B, T, V):
    return (x.transpose(0, 2, 1)
            .reshape(B, V // 8, 8, T // 128, 128)
            .transpose(0, 1, 3, 2, 4)
            .reshape(-1))


def _phys_flat_2d(x, B, T):
    return (x.reshape(B // 8, 8, T // 128, 128)
            .transpose(0, 2, 1, 3)
            .reshape(-1))


def kernel(logits, tgt_logits, actions, rewards, seq_lens):
    B, T, V = logits.shape
    sc_call = _build_sc_call(B, T, V)
    partials = sc_call(
        _phys_flat(logits, B, T, V),
        _phys_flat(tgt_logits, B, T, V),
        _phys_flat_2d(actions.astype(jnp.int32), B, T),
        seq_lens.astype(jnp.int32),
        rewards.astype(jnp.float32),
    )
    return jnp.sum(partials)


# speedup vs baseline: 1.0201x; 1.0201x over previous
import functools

import jax
import jax.numpy as jnp
from jax import lax
from jax.experimental import pallas as pl
from jax.experimental.pallas import tpu as pltpu
from jax.experimental.pallas import tpu_sc as plsc

_NC = 2
_NS = 16
_L = 16
_NW = _NC * _NS


@functools.lru_cache(maxsize=None)
def _build_sc_call(B, T, V):
    BT = B * T
    CHUNK = BT // _NW
    NCH = CHUNK // _L
    NG = CHUNK // 128
    assert BT % _NW == 0 and CHUNK % 128 == 0 and T % CHUNK == 0
    assert V % 8 == 0 and T % 128 == 0 and B % 8 == 0 and B <= _L
    NAW = CHUNK // 128 + 1
    ACT_LOAD = NAW * 128
    NTW = T // 128
    TSH = (8 * T).bit_length() - 1

    mesh = plsc.VectorSubcoreMesh(core_axis_name="c", subcore_axis_name="s")

    @functools.partial(
        pl.kernel,
        mesh=mesh,
        compiler_params=pltpu.CompilerParams(needs_layout_passes=False),
        out_type=jax.ShapeDtypeStruct((_NW, _L), jnp.float32),
        scratch_types=[
            pltpu.VMEM((ACT_LOAD,), jnp.int32),
            pltpu.VMEM((NG, 128), jnp.int32),
            pltpu.VMEM((_L,), jnp.int32),
            pltpu.VMEM((CHUNK + _L,), jnp.float32),
            pltpu.VMEM((CHUNK + _L,), jnp.float32),
            pltpu.VMEM((_L,), jnp.float32),
            pltpu.VMEM((_L,), jnp.int32),
            pltpu.VMEM((_L,), jnp.float32),
            pltpu.VMEM((_L,), jnp.float32),
            pltpu.SemaphoreType.DMA,
            pltpu.SemaphoreType.DMA,
            pltpu.SemaphoreType.DMA,
        ],
    )
    def sc_call(matq_hbm, matt_hbm, act_hbm, sl_hbm, rw_hbm, out_hbm,
                act_v, offs_v, offx_v, q_v, qt_v, xtr_v, sl_v, rw_v,
                part_v, semq, semt, sema):
        cid = lax.axis_index("c")
        sid = lax.axis_index("s")
        wid = sid * _NC + cid
        base = wid * CHUNK
        row = base // T
        t0 = base - row * T
        iota = lax.iota(jnp.int32, _L)

        g8 = lax.shift_right_logical(row, 3)
        s8 = jnp.bitwise_and(row, 7)
        u0 = lax.shift_right_logical(t0, 7)
        acopies = []
        for k in range(NAW):
            u = jnp.minimum(u0 + k, NTW - 1)
            src = ((g8 * NTW + u) * 8 + s8) * 128
            acopies.append(pltpu.async_copy(
                act_hbm.at[pl.ds(src, 128)],
                act_v.at[pl.ds(k * 128, 128)], sema))
        pltpu.sync_copy(sl_hbm, sl_v)
        pltpu.sync_copy(rw_hbm, rw_v)
        for cp in acopies:
            cp.wait()

        rowbase = row * V

        def phys_off(t16, a16):
            r16 = rowbase + a16
            return (lax.shift_left(lax.shift_right_logical(r16, 3), TSH)
                    + lax.shift_left(lax.shift_right_logical(t16, 7), 10)
                    + lax.shift_left(jnp.bitwise_and(r16, 7), 7)
                    + jnp.bitwise_and(t16, 127))

        copies = []
        for c in range(NG):
            def mk_offs(jj, _, c=c):
                j = c * 8 + jj
                a16 = act_v[pl.ds(j * _L, _L)]
                t16 = t0 + j * _L + iota
                offs_v[c, pl.ds(jj * _L, _L)] = phys_off(t16, a16)
                return 0
            lax.fori_loop(0, 8, mk_offs, 0)
            copies.append(pltpu.async_copy(
                matq_hbm.at[offs_v.at[c]],
                q_v.at[pl.ds(c * 128, 128)], semq))
            copies.append(pltpu.async_copy(
                matt_hbm.at[offs_v.at[c]],
                qt_v.at[pl.ds(c * 128, 128)], semt))

        a_x16 = act_v[pl.ds(CHUNK, _L)]
        t_x16 = jnp.minimum(t0 + CHUNK + iota, T - 1)
        offx_v[...] = phys_off(t_x16, a_x16)
        copies.append(pltpu.async_copy(matt_hbm.at[offx_v], xtr_v, semt))
        for cp in copies:
            cp.wait()
        plsc.store_scatter(qt_v, [iota * 0 + CHUNK], xtr_v[...],
                           mask=iota == 0)

        row16 = iota * 0 + row
        slr = plsc.load_gather(sl_v, [row16])
        rwr = plsc.load_gather(rw_v, [row16])
        tposv = jnp.where(slr == 0, T - 1, slr - 1)

        def acc_step(j, acc, last):
            q16 = q_v[pl.ds(j * _L, _L)]
            qtn16 = qt_v[pl.ds(j * _L + 1, _L)]
            t16 = t0 + j * _L + iota
            qb = qtn16
            if last:
                qb = jnp.where(t16 == T - 1, jnp.float32(0.0), qb)
            qb = jnp.where(t16 == tposv, rwr, qb)
            qb = jnp.minimum(jnp.maximum(qb, jnp.float32(-1.0)),
                             jnp.float32(0.0))
            d = q16 - qb
            return acc + d * d

        acc = lax.fori_loop(
            0, NCH - 1, lambda j, a: acc_step(j, a, False),
            jnp.zeros((_L,), jnp.float32))
        acc = acc_step(NCH - 1, acc, True)
        part_v[...] = acc
        pltpu.sync_copy(part_v, out_hbm.at[wid])

    return sc_call


def _phys_flat(x, B, T, V):
    return (x.transpose(0, 2, 1)
            .reshape(B, V // 8, 8, T // 128, 128)
            .transpose(0, 1, 3, 2, 4)
            .reshape(-1))


def _phys_flat_2d(x, B, T):
    return (x.reshape(B // 8, 8, T // 128, 128)
            .transpose(0, 2, 1, 3)
            .reshape(-1))


def kernel(logits, tgt_logits, actions, rewards, seq_lens):
    B, T, V = logits.shape
    sc_call = _build_sc_call(B, T, V)
    partials = sc_call(
        _phys_flat(logits, B, T, V),
        _phys_flat(tgt_logits, B, T, V),
        _phys_flat_2d(actions.astype(jnp.int32), B, T),
        seq_lens.astype(jnp.int32),
        rewards.astype(jnp.float32),
    )
    return jnp.sum(partials)


# speedup vs baseline: 1.0281x; 1.0078x over previous
import functools

import jax
import jax.numpy as jnp
from jax import lax
from jax.experimental import pallas as pl
from jax.experimental.pallas import tpu as pltpu
from jax.experimental.pallas import tpu_sc as plsc

_NC = 2
_NS = 16
_L = 16
_NW = _NC * _NS


@functools.lru_cache(maxsize=None)
def _build_sc_call(B, T, V):
    BT = B * T
    CHUNK = BT // _NW
    NCH = CHUNK // _L
    NG = CHUNK // 128
    assert BT % _NW == 0 and CHUNK % 128 == 0 and T % CHUNK == 0
    assert V % 8 == 0 and T % 128 == 0 and B % 8 == 0 and B <= _L
    NAW = CHUNK // 128 + 1
    ACT_LOAD = NAW * 128
    NTW = T // 128
    TSH = (8 * T).bit_length() - 1

    mesh = plsc.VectorSubcoreMesh(core_axis_name="c", subcore_axis_name="s")

    @functools.partial(
        pl.kernel,
        mesh=mesh,
        compiler_params=pltpu.CompilerParams(needs_layout_passes=False),
        out_type=jax.ShapeDtypeStruct((_NW, _L), jnp.float32),
        scratch_types=[
            pltpu.VMEM((ACT_LOAD,), jnp.int32),
            pltpu.VMEM((NG, 128), jnp.int32),
            pltpu.VMEM((_L,), jnp.int32),
            pltpu.VMEM((CHUNK + _L,), jnp.float32),
            pltpu.VMEM((CHUNK + _L,), jnp.float32),
            pltpu.VMEM((_L,), jnp.float32),
            pltpu.VMEM((_L,), jnp.int32),
            pltpu.VMEM((_L,), jnp.float32),
            pltpu.VMEM((_L,), jnp.float32),
            pltpu.SemaphoreType.DMA,
            pltpu.SemaphoreType.DMA,
            pltpu.SemaphoreType.DMA,
        ],
    )
    def sc_call(matq_hbm, matt_hbm, act_hbm, sl_hbm, rw_hbm, out_hbm,
                act_v, offs_v, offx_v, q_v, qt_v, xtr_v, sl_v, rw_v,
                part_v, semq, semt, sema):
        cid = lax.axis_index("c")
        sid = lax.axis_index("s")
        wid = sid * _NC + cid
        base = wid * CHUNK
        row = base // T
        t0 = base - row * T
        iota = lax.iota(jnp.int32, _L)

        g8 = lax.shift_right_logical(row, 3)
        s8 = jnp.bitwise_and(row, 7)
        u0 = lax.shift_right_logical(t0, 7)
        acopies = []
        for k in range(NAW):
            u = jnp.minimum(u0 + k, NTW - 1)
            src = ((g8 * NTW + u) * 8 + s8) * 128
            acopies.append(pltpu.async_copy(
                act_hbm.at[pl.ds(src, 128)],
                act_v.at[pl.ds(k * 128, 128)], sema))
        pltpu.sync_copy(sl_hbm, sl_v)
        pltpu.sync_copy(rw_hbm, rw_v)
        for cp in acopies:
            cp.wait()

        rowbase = row * V

        def phys_off(t16, a16):
            r16 = rowbase + a16
            return (lax.shift_left(lax.shift_right_logical(r16, 3), TSH)
                    + lax.shift_left(lax.shift_right_logical(t16, 7), 10)
                    + lax.shift_left(jnp.bitwise_and(r16, 7), 7)
                    + jnp.bitwise_and(t16, 127))

        def mk_offs(j, _):
            a16 = act_v[pl.ds(j * _L, _L)]
            t16 = t0 + j * _L + iota
            offs_v[j // 8, pl.ds((j % 8) * _L, _L)] = phys_off(t16, a16)
            return 0
        lax.fori_loop(0, NCH, mk_offs, 0)

        a_x16 = act_v[pl.ds(CHUNK, _L)]
        t_x16 = jnp.minimum(t0 + CHUNK + iota, T - 1)
        offx_v[...] = phys_off(t_x16, a_x16)

        copies = []
        for c in range(NG):
            copies.append(pltpu.async_copy(
                matq_hbm.at[offs_v.at[c]],
                q_v.at[pl.ds(c * 128, 128)], semq))
            copies.append(pltpu.async_copy(
                matt_hbm.at[offs_v.at[c]],
                qt_v.at[pl.ds(c * 128, 128)], semt))
        copies.append(pltpu.async_copy(matt_hbm.at[offx_v], xtr_v, semt))
        for cp in copies:
            cp.wait()
        plsc.store_scatter(qt_v, [iota * 0 + CHUNK], xtr_v[...],
                           mask=iota == 0)

        row16 = iota * 0 + row
        slr = plsc.load_gather(sl_v, [row16])
        rwr = plsc.load_gather(rw_v, [row16])
        tposv = jnp.where(slr == 0, T - 1, slr - 1)

        def acc_step(j, acc, last):
            q16 = q_v[pl.ds(j * _L, _L)]
            qtn16 = qt_v[pl.ds(j * _L + 1, _L)]
            t16 = t0 + j * _L + iota
            qb = qtn16
            if last:
                qb = jnp.where(t16 == T - 1, jnp.float32(0.0), qb)
            qb = jnp.where(t16 == tposv, rwr, qb)
            qb = jnp.minimum(jnp.maximum(qb, jnp.float32(-1.0)),
                             jnp.float32(0.0))
            d = q16 - qb
            return acc + d * d

        acc = lax.fori_loop(
            0, NCH - 1, lambda j, a: acc_step(j, a, False),
            jnp.zeros((_L,), jnp.float32))
        acc = acc_step(NCH - 1, acc, True)
        part_v[...] = acc
        pltpu.sync_copy(part_v, out_hbm.at[wid])

    return sc_call


def _phys_flat(x, B, T, V):
    return (x.transpose(0, 2, 1)
            .reshape(B, V // 8, 8, T // 128, 128)
            .transpose(0, 1, 3, 2, 4)
            .reshape(-1))


def _phys_flat_2d(x, B, T):
    return (x.reshape(B // 8, 8, T // 128, 128)
            .transpose(0, 2, 1, 3)
            .reshape(-1))


def kernel(logits, tgt_logits, actions, rewards, seq_lens):
    B, T, V = logits.shape
    sc_call = _build_sc_call(B, T, V)
    partials = sc_call(
        _phys_flat(logits, B, T, V),
        _phys_flat(tgt_logits, B, T, V),
        _phys_flat_2d(actions.astype(jnp.int32), B, T),
        seq_lens.astype(jnp.int32),
        rewards.astype(jnp.float32),
    )
    return jnp.sum(partials)
